# baseline (speedup 1.0000x reference)
import jax
import jax.numpy as jnp
from jax.experimental import pallas as pl

N = 10000
G = 16


def _layer_norm(h, g, b):
    mu = jnp.mean(h, axis=-1, keepdims=True)
    var = jnp.var(h, axis=-1, keepdims=True)
    return (h - mu) / jnp.sqrt(var + 1e-5) * g + b


def _cls_body(p_ref, w1_ref, b1_ref, w2_ref, b2_ref, o_ref):
    h = jnp.maximum(p_ref[...] @ w1_ref[...] + b1_ref[...], 0.0)
    o_ref[...] = h @ w2_ref[...] + b2_ref[...]


def kernel(x, edge_index, batch, W0a, b0a, W0b, b0b, ln0_g, ln0_b, W1a, b1a, W1b, b1b, ln1_g, ln1_b, Wg1, bg1, Wg2, bg2, Wc1, bc1, Wc2, bc2):
    src = edge_index[0]
    dst = edge_index[1]
    agg = jax.ops.segment_sum(x[src], dst, num_segments=N)
    h = x + agg
    h = jnp.maximum(h @ W0a + b0a, 0.0) @ W0b + b0b
    h = _layer_norm(h, ln0_g, ln0_b)
    h = jnp.maximum(h, 0.0)
    agg = jax.ops.segment_sum(h[src], dst, num_segments=N)
    h2 = h + agg
    h2 = jnp.maximum(h2 @ W1a + b1a, 0.0) @ W1b + b1b
    h2 = _layer_norm(h2, ln1_g, ln1_b)
    h2 = jnp.maximum(h2, 0.0)
    gate = (jnp.maximum(h2 @ Wg1 + bg1, 0.0) @ Wg2 + bg2)[:, 0]
    gmax = jax.ops.segment_max(gate, batch, num_segments=G)
    e = jnp.exp(gate - gmax[batch])
    den = jax.ops.segment_sum(e, batch, num_segments=G)
    alpha = e / den[batch]
    pooled = jax.ops.segment_sum(alpha[:, None] * h2, batch, num_segments=G)
    out = pl.pallas_call(
        _cls_body,
        out_shape=jax.ShapeDtypeStruct((G, 2), jnp.float32),
    )(pooled, Wc1, bc1[None, :], Wc2, bc2[None, :])
    return out


# speedup vs baseline: 6.7609x; 6.7609x over previous
import functools

import jax
import jax.numpy as jnp
from jax import lax
from jax.experimental import pallas as pl
from jax.experimental.pallas import tpu as pltpu
from jax.experimental.pallas import tpu_sc as plsc

N = 10000
E = 320000
D = 128
G = 16

NC = 2
NS = 16
NW = NC * NS
EPW = E // NW
CH = 80
NCHUNK = EPW // CH
NPAD = 10240
RPT = NPAD // NS
ZR = 32

_sc_mesh = plsc.VectorSubcoreMesh(core_axis_name="c", subcore_axis_name="s")


@functools.partial(
    pl.kernel,
    out_type=jax.ShapeDtypeStruct((NC, NPAD, D), jnp.float32),
    mesh=_sc_mesh,
    scratch_types=[
        pltpu.VMEM((NCHUNK, CH), jnp.int32),
        pltpu.VMEM((NCHUNK, CH), jnp.int32),
        pltpu.VMEM((CH, D), jnp.float32),
        pltpu.VMEM((ZR, D), jnp.float32),
        pltpu.VMEM_SHARED((NPAD, D), jnp.float32),
        pltpu.SemaphoreType.DMA,
    ],
)
def _segsum_sc(x_hbm, ei_hbm, out_hbm, src_v, dst_v, rows_v, zbuf, acc_sh, sem):
    c = lax.axis_index("c")
    s = lax.axis_index("s")
    wid = c * NS + s

    zeros16 = jnp.zeros((16,), jnp.float32)
    for i in range(ZR):
        for j in range(D // 16):
            zbuf[i, pl.ds(j * 16, 16)] = zeros16

    def zero_body(i, carry):
        pltpu.sync_copy(zbuf, acc_sh.at[pl.ds(s * RPT + i * ZR, ZR)])
        return carry

    lax.fori_loop(0, RPT // ZR, zero_body, 0)

    pltpu.sync_copy(ei_hbm.at[0, wid], src_v)
    pltpu.sync_copy(ei_hbm.at[1, wid], dst_v)

    plsc.subcore_barrier()

    def chunk_body(k, carry):
        pltpu.async_copy(x_hbm.at[src_v.at[k]], rows_v, sem).wait()
        pltpu.sync_copy(rows_v, acc_sh.at[dst_v.at[k]], add=True)
        return carry

    lax.fori_loop(0, NCHUNK, chunk_body, 0)

    plsc.subcore_barrier()

    pltpu.sync_copy(acc_sh.at[pl.ds(s * RPT, RPT)],
                    out_hbm.at[c, pl.ds(s * RPT, RPT)])


def _dense_body(hb, a0, a1, wa, ba, wb, bb, g, be, o):
    h = hb[...] + a0[...] + a1[...]
    h = jnp.maximum(jnp.dot(h, wa[...], preferred_element_type=jnp.float32)
                    + ba[...], 0.0)
    h = jnp.dot(h, wb[...], preferred_element_type=jnp.float32) + bb[...]
    mu = jnp.mean(h, axis=-1, keepdims=True)
    var = jnp.mean((h - mu) * (h - mu), axis=-1, keepdims=True)
    h = (h - mu) * lax.rsqrt(var + 1e-5) * g[...] + be[...]
    o[...] = jnp.maximum(h, 0.0)


_BN = 1000


def _dense_layer(hb, a0, a1, wa, ba, wb, bb, g, be):
    row_bs = pl.BlockSpec((_BN, D), lambda i: (i, 0))
    w_bs = pl.BlockSpec((D, D), lambda i: (0, 0))
    b_bs = pl.BlockSpec((1, D), lambda i: (0, 0))
    return pl.pallas_call(
        _dense_body,
        grid=(N // _BN,),
        in_specs=[row_bs, row_bs, row_bs, w_bs, b_bs, w_bs, b_bs, b_bs, b_bs],
        out_specs=row_bs,
        out_shape=jax.ShapeDtypeStruct((N, D), jnp.float32),
    )(hb, a0, a1, wa, ba[None, :], wb, bb[None, :], g[None, :], be[None, :])


def _pool_body(h_ref, b_ref, wg1, bg1, wg2, bg2, wc1, bc1, wc2, bc2, o_ref):
    h = h_ref[...]
    g1 = jnp.maximum(jnp.dot(h, wg1[...], preferred_element_type=jnp.float32)
                     + bg1[...], 0.0)
    gate = (jnp.dot(g1, wg2[...], preferred_element_type=jnp.float32)
            + bg2[...])[:, 0:1]
    onehot = b_ref[...] == lax.broadcasted_iota(jnp.int32, (N, G), 1)
    gmax = jnp.max(jnp.where(onehot, gate, -3e38), axis=0, keepdims=True)
    gmax_pn = jnp.sum(jnp.where(onehot, gmax, 0.0), axis=1, keepdims=True)
    e = jnp.exp(gate - gmax_pn)
    den = jnp.sum(jnp.where(onehot, e, 0.0), axis=0, keepdims=True)
    den_pn = jnp.sum(jnp.where(onehot, den, 0.0), axis=1, keepdims=True)
    alpha = e / den_pn
    w_oh = jnp.where(onehot, alpha, 0.0)
    pooled_t = lax.dot_general(h, w_oh, (((0,), (0,)), ((), ())),
                               preferred_element_type=jnp.float32)
    c1 = lax.dot_general(pooled_t, wc1[...], (((0,), (0,)), ((), ())),
                         preferred_element_type=jnp.float32)
    c1 = jnp.maximum(c1 + bc1[...], 0.0)
    o_ref[...] = (jnp.dot(c1, wc2[...], preferred_element_type=jnp.float32)
                  + bc2[...])


def _pool_classify(h, batch, Wg1, bg1, Wg2, bg2, Wc1, bc1, Wc2, bc2):
    wg2p = jnp.zeros((D, D), jnp.float32).at[:, 0].set(Wg2[:, 0])
    bg2p = jnp.zeros((1, D), jnp.float32).at[0, 0].set(bg2[0])
    wc2p = jnp.zeros((D, D), jnp.float32).at[:, :2].set(Wc2)
    bc2p = jnp.zeros((1, D), jnp.float32).at[0, :2].set(bc2)
    out = pl.pallas_call(
        _pool_body,
        out_shape=jax.ShapeDtypeStruct((G, D), jnp.float32),
    )(h, batch[:, None].astype(jnp.int32), Wg1, bg1[None, :], wg2p, bg2p,
      Wc1, bc1[None, :], wc2p, bc2p)
    return out[:, :2]


def kernel(x, edge_index, batch, W0a, b0a, W0b, b0b, ln0_g, ln0_b, W1a, b1a, W1b, b1b, ln1_g, ln1_b, Wg1, bg1, Wg2, bg2, Wc1, bc1, Wc2, bc2):
    ei_r = edge_index.reshape(2, NW, NCHUNK, CH)
    agg = _segsum_sc(x, ei_r)
    h = _dense_layer(x, agg[0, :N], agg[1, :N], W0a, b0a, W0b, b0b, ln0_g, ln0_b)
    agg = _segsum_sc(h, ei_r)
    h = _dense_layer(h, agg[0, :N], agg[1, :N], W1a, b1a, W1b, b1b, ln1_g, ln1_b)
    return _pool_classify(h, batch, Wg1, bg1, Wg2, bg2, Wc1, bc1, Wc2, bc2)


# speedup vs baseline: 10.1325x; 1.4987x over previous
import functools

import jax
import jax.numpy as jnp
from jax import lax
from jax.experimental import pallas as pl
from jax.experimental.pallas import tpu as pltpu
from jax.experimental.pallas import tpu_sc as plsc

N = 10000
E = 320000
D = 128
G = 16

NC = 2
NS = 16
NW = NC * NS
EPW = E // NW
CH = 80
NCHUNK = EPW // CH
NPAD = 10240
RPT = NPAD // NS
ZR = 16
NB = 3

_sc_mesh = plsc.VectorSubcoreMesh(core_axis_name="c", subcore_axis_name="s")


@functools.partial(
    pl.kernel,
    out_type=jax.ShapeDtypeStruct((NC, NPAD, D), jnp.float32),
    mesh=_sc_mesh,
    scratch_types=[
        [pltpu.VMEM((CH, D), jnp.float32)] * NB,
        [pltpu.VMEM((2, CH), jnp.int32)] * NB,
        pltpu.VMEM((ZR, D), jnp.float32),
        pltpu.VMEM_SHARED((NPAD, D), jnp.float32),
        pltpu.SemaphoreType.DMA((NB,)),
        pltpu.SemaphoreType.DMA((NB,)),
        pltpu.SemaphoreType.DMA((NB,)),
        pltpu.SemaphoreType.DMA,
    ],
)
def _segsum_sc(x_hbm, ei_hbm, out_hbm, bufs, islots, zbuf, acc_sh,
               gsem, ssem, isem, zsem):
    c = lax.axis_index("c")
    s = lax.axis_index("s")
    wid = c * NS + s

    zeros16 = jnp.zeros((16,), jnp.float32)
    for i in range(ZR):
        for j in range(D // 16):
            zbuf[i, pl.ds(j * 16, 16)] = zeros16

    def zero_issue(i, carry):
        pltpu.async_copy(zbuf, acc_sh.at[pl.ds(s * RPT + i * ZR, ZR)], zsem)
        return carry

    lax.fori_loop(0, RPT // ZR, zero_issue, 0)

    def i_issue(k, b):
        pltpu.async_copy(ei_hbm.at[wid, k], islots[b], isem.at[b])

    def i_wait(b):
        pltpu.make_async_copy(ei_hbm.at[0, 0], islots[b], isem.at[b]).wait()

    def g_issue(b):
        pltpu.async_copy(x_hbm.at[islots[b].at[0]], bufs[b], gsem.at[b])

    def g_wait(b):
        pltpu.make_async_copy(x_hbm.at[pl.ds(0, CH)], bufs[b],
                              gsem.at[b]).wait()

    def s_issue(b):
        pltpu.async_copy(bufs[b], acc_sh.at[islots[b].at[1]], ssem.at[b],
                         add=True)

    def s_wait(b):
        pltpu.make_async_copy(bufs[b], acc_sh.at[pl.ds(0, CH)],
                              ssem.at[b]).wait()

    i_issue(0, 0)
    i_issue(1, 1)

    def zero_drain(i, carry):
        pltpu.make_async_copy(zbuf, acc_sh.at[pl.ds(0, ZR)], zsem).wait()
        return carry

    lax.fori_loop(0, RPT // ZR, zero_drain, 0)
    plsc.subcore_barrier()

    i_wait(0)
    g_issue(0)

    def main_body(i, carry):
        for p in range(NB):
            j = i * NB + p
            pn = (p + 1) % NB
            pnn = (p + 2) % NB

            if p == 0:
                @pl.when(i > 0)
                def _():
                    s_wait(pnn)
            else:
                s_wait(pnn)

            @pl.when(j + 2 < NCHUNK)
            def _():
                i_issue(j + 2, pnn)

            @pl.when(j + 1 < NCHUNK)
            def _():
                i_wait(pn)
                g_issue(pn)

            g_wait(p)
            s_issue(p)
        return carry

    lax.fori_loop(0, (NCHUNK - NB + 1) // NB, main_body, 0)

    s_wait(2)
    i_wait(1)
    g_issue(1)
    g_wait(0)
    s_issue(0)
    s_wait(0)
    g_wait(1)
    s_issue(1)
    s_wait(1)

    plsc.subcore_barrier()

    pltpu.sync_copy(acc_sh.at[pl.ds(s * RPT, RPT)],
                    out_hbm.at[c, pl.ds(s * RPT, RPT)])


def _dense_body(hb, a0, a1, wa, ba, wb, bb, g, be, o):
    h = hb[...] + a0[...] + a1[...]
    h = jnp.maximum(jnp.dot(h, wa[...], preferred_element_type=jnp.float32)
                    + ba[...], 0.0)
    h = jnp.dot(h, wb[...], preferred_element_type=jnp.float32) + bb[...]
    mu = jnp.mean(h, axis=-1, keepdims=True)
    var = jnp.mean((h - mu) * (h - mu), axis=-1, keepdims=True)
    h = (h - mu) * lax.rsqrt(var + 1e-5) * g[...] + be[...]
    o[...] = jnp.maximum(h, 0.0)


_BN = 1000


def _dense_layer(hb, a0, a1, wa, ba, wb, bb, g, be):
    row_bs = pl.BlockSpec((_BN, D), lambda i: (i, 0))
    w_bs = pl.BlockSpec((D, D), lambda i: (0, 0))
    b_bs = pl.BlockSpec((1, D), lambda i: (0, 0))
    return pl.pallas_call(
        _dense_body,
        grid=(N // _BN,),
        in_specs=[row_bs, row_bs, row_bs, w_bs, b_bs, w_bs, b_bs, b_bs, b_bs],
        out_specs=row_bs,
        out_shape=jax.ShapeDtypeStruct((N, D), jnp.float32),
    )(hb, a0, a1, wa, ba[None, :], wb, bb[None, :], g[None, :], be[None, :])


def _pool_body(h_ref, b_ref, wg1, bg1, wg2, bg2, wc1, bc1, wc2, bc2, o_ref):
    h = h_ref[...]
    g1 = jnp.maximum(jnp.dot(h, wg1[...], preferred_element_type=jnp.float32)
                     + bg1[...], 0.0)
    gate = (jnp.dot(g1, wg2[...], preferred_element_type=jnp.float32)
            + bg2[...])[:, 0:1]
    onehot = b_ref[...] == lax.broadcasted_iota(jnp.int32, (N, G), 1)
    gmax = jnp.max(jnp.where(onehot, gate, -3e38), axis=0, keepdims=True)
    gmax_pn = jnp.sum(jnp.where(onehot, gmax, 0.0), axis=1, keepdims=True)
    e = jnp.exp(gate - gmax_pn)
    den = jnp.sum(jnp.where(onehot, e, 0.0), axis=0, keepdims=True)
    den_pn = jnp.sum(jnp.where(onehot, den, 0.0), axis=1, keepdims=True)
    alpha = e / den_pn
    w_oh = jnp.where(onehot, alpha, 0.0)
    pooled_t = lax.dot_general(h, w_oh, (((0,), (0,)), ((), ())),
                               preferred_element_type=jnp.float32)
    c1 = lax.dot_general(pooled_t, wc1[...], (((0,), (0,)), ((), ())),
                         preferred_element_type=jnp.float32)
    c1 = jnp.maximum(c1 + bc1[...], 0.0)
    o_ref[...] = (jnp.dot(c1, wc2[...], preferred_element_type=jnp.float32)
                  + bc2[...])


def _pool_classify(h, batch, Wg1, bg1, Wg2, bg2, Wc1, bc1, Wc2, bc2):
    wg2p = jnp.zeros((D, D), jnp.float32).at[:, 0].set(Wg2[:, 0])
    bg2p = jnp.zeros((1, D), jnp.float32).at[0, 0].set(bg2[0])
    wc2p = jnp.zeros((D, D), jnp.float32).at[:, :2].set(Wc2)
    bc2p = jnp.zeros((1, D), jnp.float32).at[0, :2].set(bc2)
    out = pl.pallas_call(
        _pool_body,
        out_shape=jax.ShapeDtypeStruct((G, D), jnp.float32),
    )(h, batch[:, None].astype(jnp.int32), Wg1, bg1[None, :], wg2p, bg2p,
      Wc1, bc1[None, :], wc2p, bc2p)
    return out[:, :2]


def kernel(x, edge_index, batch, W0a, b0a, W0b, b0b, ln0_g, ln0_b, W1a, b1a, W1b, b1b, ln1_g, ln1_b, Wg1, bg1, Wg2, bg2, Wc1, bc1, Wc2, bc2):
    eip = jnp.stack([edge_index[0].reshape(NW, NCHUNK, CH),
                     edge_index[1].reshape(NW, NCHUNK, CH)], axis=2)
    agg = _segsum_sc(x, eip)
    h = _dense_layer(x, agg[0, :N], agg[1, :N], W0a, b0a, W0b, b0b, ln0_g, ln0_b)
    agg = _segsum_sc(h, eip)
    h = _dense_layer(h, agg[0, :N], agg[1, :N], W1a, b1a, W1b, b1b, ln1_g, ln1_b)
    return _pool_classify(h, batch, Wg1, bg1, Wg2, bg2, Wc1, bc1, Wc2, bc2)


# speedup vs baseline: 11.3512x; 1.1203x over previous
import functools

import jax
import jax.numpy as jnp
from jax import lax
from jax.experimental import pallas as pl
from jax.experimental.pallas import tpu as pltpu
from jax.experimental.pallas import tpu_sc as plsc

N = 10000
E = 320000
D = 128
G = 16

NC = 2
NS = 16
NW = NC * NS
EPW = E // NW
CH = 80
NCHUNK = EPW // CH
NPAD = 10240
RPT = NPAD // NS
ZR = 16
NB = 3

_sc_mesh = plsc.VectorSubcoreMesh(core_axis_name="c", subcore_axis_name="s")


@functools.partial(
    pl.kernel,
    out_type=jax.ShapeDtypeStruct((NC, NPAD, D), jnp.float32),
    mesh=_sc_mesh,
    scratch_types=[
        [pltpu.VMEM((CH, D), jnp.float32)] * NB,
        [pltpu.VMEM((2, CH), jnp.int32)] * NB,
        pltpu.VMEM((ZR, D), jnp.float32),
        pltpu.VMEM_SHARED((NPAD, D), jnp.float32),
        pltpu.SemaphoreType.DMA((NB,)),
        pltpu.SemaphoreType.DMA((NB,)),
        pltpu.SemaphoreType.DMA((NB,)),
        pltpu.SemaphoreType.DMA,
    ],
)
def _segsum_sc(x_hbm, ei_hbm, out_hbm, bufs, islots, zbuf, acc_sh,
               gsem, ssem, isem, zsem):
    c = lax.axis_index("c")
    s = lax.axis_index("s")
    wid = c * NS + s

    zeros16 = jnp.zeros((16,), jnp.float32)
    for i in range(ZR):
        for j in range(D // 16):
            zbuf[i, pl.ds(j * 16, 16)] = zeros16

    def zero_issue(i, carry):
        pltpu.async_copy(zbuf, acc_sh.at[pl.ds(s * RPT + i * ZR, ZR)], zsem)
        return carry

    lax.fori_loop(0, RPT // ZR, zero_issue, 0)

    def i_issue(k, b):
        pltpu.async_copy(ei_hbm.at[0, wid, k], islots[b].at[0], isem.at[b])
        pltpu.async_copy(ei_hbm.at[1, wid, k], islots[b].at[1], isem.at[b])

    def i_wait(b):
        pltpu.make_async_copy(ei_hbm.at[0, 0, 0], islots[b].at[0],
                              isem.at[b]).wait()
        pltpu.make_async_copy(ei_hbm.at[0, 0, 0], islots[b].at[1],
                              isem.at[b]).wait()

    def g_issue(b):
        pltpu.async_copy(x_hbm.at[islots[b].at[0]], bufs[b], gsem.at[b])

    def g_wait(b):
        pltpu.make_async_copy(x_hbm.at[pl.ds(0, CH)], bufs[b],
                              gsem.at[b]).wait()

    def s_issue(b):
        pltpu.async_copy(bufs[b], acc_sh.at[islots[b].at[1]], ssem.at[b],
                         add=True)

    def s_wait(b):
        pltpu.make_async_copy(bufs[b], acc_sh.at[pl.ds(0, CH)],
                              ssem.at[b]).wait()

    i_issue(0, 0)
    i_issue(1, 1)

    def zero_drain(i, carry):
        pltpu.make_async_copy(zbuf, acc_sh.at[pl.ds(0, ZR)], zsem).wait()
        return carry

    lax.fori_loop(0, RPT // ZR, zero_drain, 0)
    plsc.subcore_barrier()

    i_wait(0)
    g_issue(0)

    def main_body(i, carry):
        for p in range(NB):
            j = i * NB + p
            pn = (p + 1) % NB
            pnn = (p + 2) % NB

            if p == 0:
                @pl.when(i > 0)
                def _():
                    s_wait(pnn)
            else:
                s_wait(pnn)

            @pl.when(j + 2 < NCHUNK)
            def _():
                i_issue(j + 2, pnn)

            @pl.when(j + 1 < NCHUNK)
            def _():
                i_wait(pn)
                g_issue(pn)

            g_wait(p)
            s_issue(p)
        return carry

    lax.fori_loop(0, (NCHUNK - NB + 1) // NB, main_body, 0)

    s_wait(2)
    i_wait(1)
    g_issue(1)
    g_wait(0)
    s_issue(0)
    s_wait(0)
    g_wait(1)
    s_issue(1)
    s_wait(1)

    plsc.subcore_barrier()

    pltpu.sync_copy(acc_sh.at[pl.ds(s * RPT, RPT)],
                    out_hbm.at[c, pl.ds(s * RPT, RPT)])


def _dense_body(hb, ag, wa, ba, wb, bb, g, be, o):
    h = hb[...] + ag[0] + ag[1]
    h = jnp.maximum(jnp.dot(h, wa[...], preferred_element_type=jnp.float32)
                    + ba[...], 0.0)
    h = jnp.dot(h, wb[...], preferred_element_type=jnp.float32) + bb[...]
    mu = jnp.mean(h, axis=-1, keepdims=True)
    var = jnp.mean((h - mu) * (h - mu), axis=-1, keepdims=True)
    h = (h - mu) * lax.rsqrt(var + 1e-5) * g[...] + be[...]
    o[...] = jnp.maximum(h, 0.0)


_BN = 1000


def _dense_layer(hb, ag, wa, ba, wb, bb, g, be):
    row_bs = pl.BlockSpec((_BN, D), lambda i: (i, 0))
    agg_bs = pl.BlockSpec((NC, _BN, D), lambda i: (0, i, 0))
    w_bs = pl.BlockSpec((D, D), lambda i: (0, 0))
    b_bs = pl.BlockSpec((1, D), lambda i: (0, 0))
    return pl.pallas_call(
        _dense_body,
        grid=(N // _BN,),
        in_specs=[row_bs, agg_bs, w_bs, b_bs, w_bs, b_bs, b_bs, b_bs],
        out_specs=row_bs,
        out_shape=jax.ShapeDtypeStruct((N, D), jnp.float32),
    )(hb, ag, wa, ba[None, :], wb, bb[None, :], g[None, :], be[None, :])


def _pool_body(h_ref, b_ref, wg1, bg1, wg2, bg2, wc1, bc1, wc2, bc2, o_ref):
    h = h_ref[...]
    g1 = jnp.maximum(jnp.dot(h, wg1[...], preferred_element_type=jnp.float32)
                     + bg1[...], 0.0)
    gate = (jnp.dot(g1, wg2[...], preferred_element_type=jnp.float32)
            + bg2[...])[:, 0:1]
    onehot = b_ref[...] == lax.broadcasted_iota(jnp.int32, (N, G), 1)
    gmax = jnp.max(jnp.where(onehot, gate, -3e38), axis=0, keepdims=True)
    gmax_pn = jnp.sum(jnp.where(onehot, gmax, 0.0), axis=1, keepdims=True)
    e = jnp.exp(gate - gmax_pn)
    den = jnp.sum(jnp.where(onehot, e, 0.0), axis=0, keepdims=True)
    den_pn = jnp.sum(jnp.where(onehot, den, 0.0), axis=1, keepdims=True)
    alpha = e / den_pn
    w_oh = jnp.where(onehot, alpha, 0.0)
    pooled_t = lax.dot_general(h, w_oh, (((0,), (0,)), ((), ())),
                               preferred_element_type=jnp.float32)
    c1 = lax.dot_general(pooled_t, wc1[...], (((0,), (0,)), ((), ())),
                         preferred_element_type=jnp.float32)
    c1 = jnp.maximum(c1 + bc1[...], 0.0)
    o_ref[...] = (jnp.dot(c1, wc2[...], preferred_element_type=jnp.float32)
                  + bc2[...])


def _pool_classify(h, batch, Wg1, bg1, Wg2, bg2, Wc1, bc1, Wc2, bc2):
    wg2p = jnp.zeros((D, D), jnp.float32).at[:, 0].set(Wg2[:, 0])
    bg2p = jnp.zeros((1, D), jnp.float32).at[0, 0].set(bg2[0])
    wc2p = jnp.zeros((D, D), jnp.float32).at[:, :2].set(Wc2)
    bc2p = jnp.zeros((1, D), jnp.float32).at[0, :2].set(bc2)
    out = pl.pallas_call(
        _pool_body,
        out_shape=jax.ShapeDtypeStruct((G, D), jnp.float32),
    )(h, batch[:, None].astype(jnp.int32), Wg1, bg1[None, :], wg2p, bg2p,
      Wc1, bc1[None, :], wc2p, bc2p)
    return out[:, :2]


def kernel(x, edge_index, batch, W0a, b0a, W0b, b0b, ln0_g, ln0_b, W1a, b1a, W1b, b1b, ln1_g, ln1_b, Wg1, bg1, Wg2, bg2, Wc1, bc1, Wc2, bc2):
    eip = edge_index.reshape(2, NW, NCHUNK, CH)
    agg = _segsum_sc(x, eip)
    h = _dense_layer(x, agg, W0a, b0a, W0b, b0b, ln0_g, ln0_b)
    agg = _segsum_sc(h, eip)
    h = _dense_layer(h, agg, W1a, b1a, W1b, b1b, ln1_g, ln1_b)
    return _pool_classify(h, batch, Wg1, bg1, Wg2, bg2, Wc1, bc1, Wc2, bc2)


# speedup vs baseline: 11.5997x; 1.0219x over previous
import functools

import jax
import jax.numpy as jnp
from jax import lax
from jax.experimental import pallas as pl
from jax.experimental.pallas import tpu as pltpu
from jax.experimental.pallas import tpu_sc as plsc

N = 10000
E = 320000
D = 128
G = 16

NC = 2
NS = 16
NW = NC * NS
EPW = E // NW
CH = 80
NCHUNK = EPW // CH
NPAD = 10240
RPT = NPAD // NS
ZR = 16
NB = 3

_sc_mesh = plsc.VectorSubcoreMesh(core_axis_name="c", subcore_axis_name="s")


@functools.partial(
    pl.kernel,
    out_type=jax.ShapeDtypeStruct((NC, NPAD, D), jnp.float32),
    mesh=_sc_mesh,
    scratch_types=[
        [pltpu.VMEM((CH, D), jnp.float32)] * NB,
        [pltpu.VMEM((2, CH), jnp.int32)] * NB,
        pltpu.VMEM((ZR, D), jnp.float32),
        pltpu.VMEM_SHARED((NPAD, D), jnp.float32),
        pltpu.SemaphoreType.DMA((NB,)),
        pltpu.SemaphoreType.DMA((NB,)),
        pltpu.SemaphoreType.DMA((NB,)),
        pltpu.SemaphoreType.DMA,
    ],
)
def _segsum_sc(x_hbm, ei_hbm, out_hbm, bufs, islots, zbuf, acc_sh,
               gsem, ssem, isem, zsem):
    c = lax.axis_index("c")
    s = lax.axis_index("s")
    wid = c * NS + s

    zeros16 = jnp.zeros((16,), jnp.float32)
    for i in range(ZR):
        for j in range(D // 16):
            zbuf[i, pl.ds(j * 16, 16)] = zeros16

    def zero_issue(i, carry):
        pltpu.async_copy(zbuf, acc_sh.at[pl.ds(s * RPT + i * ZR, ZR)], zsem)
        return carry

    lax.fori_loop(0, RPT // ZR, zero_issue, 0)

    def i_issue(k, b):
        pltpu.async_copy(ei_hbm.at[0, wid, k], islots[b].at[0], isem.at[b])
        pltpu.async_copy(ei_hbm.at[1, wid, k], islots[b].at[1], isem.at[b])

    def i_wait(b):
        pltpu.make_async_copy(ei_hbm.at[0, 0, 0], islots[b].at[0],
                              isem.at[b]).wait()
        pltpu.make_async_copy(ei_hbm.at[0, 0, 0], islots[b].at[1],
                              isem.at[b]).wait()

    def g_issue(b):
        pltpu.async_copy(x_hbm.at[islots[b].at[0]], bufs[b], gsem.at[b])

    def g_wait(b):
        pltpu.make_async_copy(x_hbm.at[pl.ds(0, CH)], bufs[b],
                              gsem.at[b]).wait()

    def s_issue(b):
        pltpu.async_copy(bufs[b], acc_sh.at[islots[b].at[1]], ssem.at[b],
                         add=True)

    def s_wait(b):
        pltpu.make_async_copy(bufs[b], acc_sh.at[pl.ds(0, CH)],
                              ssem.at[b]).wait()

    i_issue(0, 0)
    i_issue(1, 1)

    def zero_drain(i, carry):
        pltpu.make_async_copy(zbuf, acc_sh.at[pl.ds(0, ZR)], zsem).wait()
        return carry

    i_wait(0)
    g_issue(0)
    lax.fori_loop(0, RPT // ZR, zero_drain, 0)
    plsc.subcore_barrier()

    def main_body(i, carry):
        for p in range(NB):
            j = i * NB + p
            pn = (p + 1) % NB
            pnn = (p + 2) % NB

            if p == 0:
                @pl.when(i > 0)
                def _():
                    s_wait(pnn)
            else:
                s_wait(pnn)

            @pl.when(j + 2 < NCHUNK)
            def _():
                i_issue(j + 2, pnn)

            @pl.when(j + 1 < NCHUNK)
            def _():
                i_wait(pn)
                g_issue(pn)

            g_wait(p)
            s_issue(p)
        return carry

    lax.fori_loop(0, (NCHUNK - NB + 1) // NB, main_body, 0)

    s_wait(2)
    i_wait(1)
    g_issue(1)
    g_wait(0)
    s_issue(0)
    s_wait(0)
    g_wait(1)
    s_issue(1)
    s_wait(1)

    plsc.subcore_barrier()

    pltpu.sync_copy(acc_sh.at[pl.ds(s * RPT, RPT)],
                    out_hbm.at[c, pl.ds(s * RPT, RPT)])


def _dense_body(hb, ag, wa, ba, wb, bb, g, be, o):
    h = hb[...] + ag[0] + ag[1]
    h = jnp.maximum(jnp.dot(h, wa[...], preferred_element_type=jnp.float32)
                    + ba[...], 0.0)
    h = jnp.dot(h, wb[...], preferred_element_type=jnp.float32) + bb[...]
    mu = jnp.mean(h, axis=-1, keepdims=True)
    var = jnp.mean((h - mu) * (h - mu), axis=-1, keepdims=True)
    h = (h - mu) * lax.rsqrt(var + 1e-5) * g[...] + be[...]
    o[...] = jnp.maximum(h, 0.0)


_BN = 1000


def _dense_layer(hb, ag, wa, ba, wb, bb, g, be):
    row_bs = pl.BlockSpec((_BN, D), lambda i: (i, 0))
    agg_bs = pl.BlockSpec((NC, _BN, D), lambda i: (0, i, 0))
    w_bs = pl.BlockSpec((D, D), lambda i: (0, 0))
    b_bs = pl.BlockSpec((1, D), lambda i: (0, 0))
    return pl.pallas_call(
        _dense_body,
        grid=(N // _BN,),
        in_specs=[row_bs, agg_bs, w_bs, b_bs, w_bs, b_bs, b_bs, b_bs],
        out_specs=row_bs,
        out_shape=jax.ShapeDtypeStruct((N, D), jnp.float32),
    )(hb, ag, wa, ba[None, :], wb, bb[None, :], g[None, :], be[None, :])


def _dense_pool_body(hb, ag, wa, ba, wb, bb, g, be, b_ref, wg1, bg1, wg2,
                     bg2, wc1, bc1, wc2, bc2, o_ref, h_s, gate_s):
    i = pl.program_id(0)
    h = hb[...] + ag[0] + ag[1]
    h = jnp.maximum(jnp.dot(h, wa[...], preferred_element_type=jnp.float32)
                    + ba[...], 0.0)
    h = jnp.dot(h, wb[...], preferred_element_type=jnp.float32) + bb[...]
    mu = jnp.mean(h, axis=-1, keepdims=True)
    var = jnp.mean((h - mu) * (h - mu), axis=-1, keepdims=True)
    h = (h - mu) * lax.rsqrt(var + 1e-5) * g[...] + be[...]
    h = jnp.maximum(h, 0.0)
    h_s[pl.ds(i * _BN, _BN), :] = h
    g1 = jnp.maximum(jnp.dot(h, wg1[...], preferred_element_type=jnp.float32)
                     + bg1[...], 0.0)
    gate_s[pl.ds(i * _BN, _BN), :] = (
        jnp.dot(g1, wg2[...], preferred_element_type=jnp.float32)
        + bg2[...])[:, 0:1]

    @pl.when(i == N // _BN - 1)
    def _():
        gate = gate_s[...]
        onehot = b_ref[...] == lax.broadcasted_iota(jnp.int32, (N, G), 1)
        ohf = onehot.astype(jnp.float32)
        gmax = jnp.max(jnp.where(onehot, gate, -3e38), axis=0,
                       keepdims=True)
        gmax_pn = lax.dot_general(ohf, gmax, (((1,), (1,)), ((), ())),
                                  preferred_element_type=jnp.float32)
        e = jnp.exp(gate - gmax_pn)
        den = lax.dot_general(e, ohf, (((0,), (0,)), ((), ())),
                              preferred_element_type=jnp.float32)
        den_pn = lax.dot_general(ohf, den, (((1,), (1,)), ((), ())),
                                 preferred_element_type=jnp.float32)
        w_oh = ohf * (e / den_pn)
        hf = h_s[...]
        pooled_t = lax.dot_general(hf, w_oh, (((0,), (0,)), ((), ())),
                                   preferred_element_type=jnp.float32)
        c1 = lax.dot_general(pooled_t, wc1[...], (((0,), (0,)), ((), ())),
                             preferred_element_type=jnp.float32)
        c1 = jnp.maximum(c1 + bc1[...], 0.0)
        o_ref[...] = (jnp.dot(c1, wc2[...],
                              preferred_element_type=jnp.float32) + bc2[...])


def _dense_pool(hb, ag, wa, ba, wb, bb, g, be, batch,
                Wg1, bg1, Wg2, bg2, Wc1, bc1, Wc2, bc2):
    wg2p = jnp.zeros((D, D), jnp.float32).at[:, 0].set(Wg2[:, 0])
    bg2p = jnp.zeros((1, D), jnp.float32).at[0, 0].set(bg2[0])
    wc2p = jnp.zeros((D, D), jnp.float32).at[:, :2].set(Wc2)
    bc2p = jnp.zeros((1, D), jnp.float32).at[0, :2].set(bc2)
    row_bs = pl.BlockSpec((_BN, D), lambda i: (i, 0))
    agg_bs = pl.BlockSpec((NC, _BN, D), lambda i: (0, i, 0))
    w_bs = pl.BlockSpec((D, D), lambda i: (0, 0))
    b_bs = pl.BlockSpec((1, D), lambda i: (0, 0))
    n1_bs = pl.BlockSpec((N, 1), lambda i: (0, 0))
    out = pl.pallas_call(
        _dense_pool_body,
        grid=(N // _BN,),
        in_specs=[row_bs, agg_bs, w_bs, b_bs, w_bs, b_bs, b_bs, b_bs,
                  n1_bs, w_bs, b_bs, w_bs, b_bs, w_bs, b_bs, w_bs, b_bs],
        out_specs=pl.BlockSpec((G, D), lambda i: (0, 0)),
        out_shape=jax.ShapeDtypeStruct((G, D), jnp.float32),
        scratch_shapes=[pltpu.VMEM((N, D), jnp.float32),
                        pltpu.VMEM((N, 1), jnp.float32)],
    )(hb, ag, wa, ba[None, :], wb, bb[None, :], g[None, :], be[None, :],
      batch[:, None].astype(jnp.int32), Wg1, bg1[None, :], wg2p, bg2p,
      Wc1, bc1[None, :], wc2p, bc2p)
    return out[:, :2]


def kernel(x, edge_index, batch, W0a, b0a, W0b, b0b, ln0_g, ln0_b, W1a, b1a, W1b, b1b, ln1_g, ln1_b, Wg1, bg1, Wg2, bg2, Wc1, bc1, Wc2, bc2):
    eip = edge_index.reshape(2, NW, NCHUNK, CH)
    agg = _segsum_sc(x, eip)
    h = _dense_layer(x, agg, W0a, b0a, W0b, b0b, ln0_g, ln0_b)
    agg = _segsum_sc(h, eip)
    return _dense_pool(h, agg, W1a, b1a, W1b, b1b, ln1_g, ln1_b, batch,
                       Wg1, bg1, Wg2, bg2, Wc1, bc1, Wc2, bc2)


# speedup vs baseline: 12.0562x; 1.0394x over previous
import functools

import jax
import jax.numpy as jnp
from jax import lax
from jax.experimental import pallas as pl
from jax.experimental.pallas import tpu as pltpu
from jax.experimental.pallas import tpu_sc as plsc

N = 10000
E = 320000
D = 128
G = 16

NC = 2
NS = 16
NW = NC * NS
EPW = E // NW
CH = 80
NCHUNK = EPW // CH
NPAD = 10240
RPT = NPAD // NS
ZR = 16
NB = 3
NI = 6

_sc_mesh = plsc.VectorSubcoreMesh(core_axis_name="c", subcore_axis_name="s")


@functools.partial(
    pl.kernel,
    out_type=jax.ShapeDtypeStruct((NC, NPAD, D), jnp.float32),
    mesh=_sc_mesh,
    scratch_types=[
        [pltpu.VMEM((CH, D), jnp.float32)] * NB,
        [pltpu.VMEM((2, CH), jnp.int32)] * NI,
        pltpu.VMEM((ZR, D), jnp.float32),
        pltpu.VMEM_SHARED((NPAD, D), jnp.float32),
        pltpu.SemaphoreType.DMA((NB,)),
        pltpu.SemaphoreType.DMA((NB,)),
        pltpu.SemaphoreType.DMA((NI,)),
        pltpu.SemaphoreType.DMA,
    ],
)
def _segsum_sc(x_hbm, ei_hbm, out_hbm, bufs, islots, zbuf, acc_sh,
               gsem, ssem, isem, zsem):
    c = lax.axis_index("c")
    s = lax.axis_index("s")
    wid = c * NS + s

    zeros16 = jnp.zeros((16,), jnp.float32)
    for i in range(ZR):
        for j in range(D // 16):
            zbuf[i, pl.ds(j * 16, 16)] = zeros16

    def zero_issue(i, carry):
        pltpu.async_copy(zbuf, acc_sh.at[pl.ds(s * RPT + i * ZR, ZR)], zsem)
        return carry

    lax.fori_loop(0, RPT // ZR, zero_issue, 0)

    def i_issue(k, b):
        pltpu.async_copy(ei_hbm.at[wid, k], islots[b], isem.at[b])

    def i_wait(b):
        pltpu.make_async_copy(ei_hbm.at[0, 0], islots[b], isem.at[b]).wait()

    def g_issue(q, b):
        pltpu.async_copy(x_hbm.at[islots[q].at[0]], bufs[b], gsem.at[b])

    def g_wait(b):
        pltpu.make_async_copy(x_hbm.at[pl.ds(0, CH)], bufs[b],
                              gsem.at[b]).wait()

    def s_issue(b, q):
        pltpu.async_copy(bufs[b], acc_sh.at[islots[q].at[1]], ssem.at[b],
                         add=True)

    def s_wait(b):
        pltpu.make_async_copy(bufs[b], acc_sh.at[pl.ds(0, CH)],
                              ssem.at[b]).wait()

    i_issue(0, 0)
    i_issue(1, 1)
    i_issue(2, 2)

    def zero_drain(i, carry):
        pltpu.make_async_copy(zbuf, acc_sh.at[pl.ds(0, ZR)], zsem).wait()
        return carry

    i_wait(0)
    g_issue(0, 0)
    lax.fori_loop(0, RPT // ZR, zero_drain, 0)
    plsc.subcore_barrier()

    def main_body(i, carry):
        for q in range(NI):
            j = i * NI + q
            p = q % NB
            pn = (p + 1) % NB
            qn = (q + 1) % NI

            i_issue(j + 3, (q + 3) % NI)
            i_wait(qn)
            if q < 2:
                @pl.when(i > 0)
                def _():
                    s_wait(pn)
            else:
                s_wait(pn)
            g_issue(qn, pn)
            g_wait(p)
            s_issue(p, q)
        return carry

    lax.fori_loop(0, (NCHUNK - 5) // NI, main_body, 0)

    def tail_body(i, carry):
        base = i * NI
        i_issue(base + 3, 3)
        i_wait(1)
        s_wait(1)
        g_issue(1, 1)
        g_wait(0)
        s_issue(0, 0)
        i_issue(base + 4, 4)
        i_wait(2)
        s_wait(2)
        g_issue(2, 2)
        g_wait(1)
        s_issue(1, 1)
        i_wait(3)
        s_wait(0)
        g_issue(3, 0)
        g_wait(2)
        s_issue(2, 2)
        i_wait(4)
        s_wait(1)
        g_issue(4, 1)
        g_wait(0)
        s_issue(0, 3)
        s_wait(2)
        g_wait(1)
        s_issue(1, 4)
        return carry

    lax.fori_loop(NCHUNK // NI, NCHUNK // NI + 1, tail_body, 0)
    s_wait(0)
    s_wait(1)

    plsc.subcore_barrier()

    pltpu.sync_copy(acc_sh.at[pl.ds(s * RPT, RPT)],
                    out_hbm.at[c, pl.ds(s * RPT, RPT)])


def _dense_body(hb, ag, wa, ba, wb, bb, g, be, o):
    h = hb[...] + ag[0] + ag[1]
    h = jnp.maximum(jnp.dot(h, wa[...], preferred_element_type=jnp.float32)
                    + ba[...], 0.0)
    h = jnp.dot(h, wb[...], preferred_element_type=jnp.float32) + bb[...]
    mu = jnp.mean(h, axis=-1, keepdims=True)
    var = jnp.mean((h - mu) * (h - mu), axis=-1, keepdims=True)
    h = (h - mu) * lax.rsqrt(var + 1e-5) * g[...] + be[...]
    o[...] = jnp.maximum(h, 0.0)


_BN = 1000


def _dense_layer(hb, ag, wa, ba, wb, bb, g, be):
    row_bs = pl.BlockSpec((_BN, D), lambda i: (i, 0))
    agg_bs = pl.BlockSpec((NC, _BN, D), lambda i: (0, i, 0))
    w_bs = pl.BlockSpec((D, D), lambda i: (0, 0))
    b_bs = pl.BlockSpec((1, D), lambda i: (0, 0))
    return pl.pallas_call(
        _dense_body,
        grid=(N // _BN,),
        in_specs=[row_bs, agg_bs, w_bs, b_bs, w_bs, b_bs, b_bs, b_bs],
        out_specs=row_bs,
        out_shape=jax.ShapeDtypeStruct((N, D), jnp.float32),
    )(hb, ag, wa, ba[None, :], wb, bb[None, :], g[None, :], be[None, :])


def _dense_pool_body(hb, ag, wa, ba, wb, bb, g, be, b_ref, wg1, bg1, wg2,
                     bg2, wc1, bc1, wc2, bc2, o_ref, h_s, gate_s):
    i = pl.program_id(0)
    h = hb[...] + ag[0] + ag[1]
    h = jnp.maximum(jnp.dot(h, wa[...], preferred_element_type=jnp.float32)
                    + ba[...], 0.0)
    h = jnp.dot(h, wb[...], preferred_element_type=jnp.float32) + bb[...]
    mu = jnp.mean(h, axis=-1, keepdims=True)
    var = jnp.mean((h - mu) * (h - mu), axis=-1, keepdims=True)
    h = (h - mu) * lax.rsqrt(var + 1e-5) * g[...] + be[...]
    h = jnp.maximum(h, 0.0)
    h_s[pl.ds(i * _BN, _BN), :] = h
    g1 = jnp.maximum(jnp.dot(h, wg1[...], preferred_element_type=jnp.float32)
                     + bg1[...], 0.0)
    gate_s[pl.ds(i * _BN, _BN), :] = (
        jnp.dot(g1, wg2[...], preferred_element_type=jnp.float32)
        + bg2[...])[:, 0:1]

    @pl.when(i == N // _BN - 1)
    def _():
        gate = gate_s[...]
        onehot = b_ref[...] == lax.broadcasted_iota(jnp.int32, (N, G), 1)
        ohf = onehot.astype(jnp.float32)
        gmax = jnp.max(jnp.where(onehot, gate, -3e38), axis=0,
                       keepdims=True)
        gmax_pn = lax.dot_general(ohf, gmax, (((1,), (1,)), ((), ())),
                                  preferred_element_type=jnp.float32)
        e = jnp.exp(gate - gmax_pn)
        den = lax.dot_general(e, ohf, (((0,), (0,)), ((), ())),
                              preferred_element_type=jnp.float32)
        den_pn = lax.dot_general(ohf, den, (((1,), (1,)), ((), ())),
                                 preferred_element_type=jnp.float32)
        w_oh = ohf * (e / den_pn)
        hf = h_s[...]
        pooled_t = lax.dot_general(hf, w_oh, (((0,), (0,)), ((), ())),
                                   preferred_element_type=jnp.float32)
        c1 = lax.dot_general(pooled_t, wc1[...], (((0,), (0,)), ((), ())),
                             preferred_element_type=jnp.float32)
        c1 = jnp.maximum(c1 + bc1[...], 0.0)
        o_ref[...] = (jnp.dot(c1, wc2[...],
                              preferred_element_type=jnp.float32) + bc2[...])


def _dense_pool(hb, ag, wa, ba, wb, bb, g, be, batch,
                Wg1, bg1, Wg2, bg2, Wc1, bc1, Wc2, bc2):
    wg2p = jnp.zeros((D, D), jnp.float32).at[:, 0].set(Wg2[:, 0])
    bg2p = jnp.zeros((1, D), jnp.float32).at[0, 0].set(bg2[0])
    wc2p = jnp.zeros((D, D), jnp.float32).at[:, :2].set(Wc2)
    bc2p = jnp.zeros((1, D), jnp.float32).at[0, :2].set(bc2)
    row_bs = pl.BlockSpec((_BN, D), lambda i: (i, 0))
    agg_bs = pl.BlockSpec((NC, _BN, D), lambda i: (0, i, 0))
    w_bs = pl.BlockSpec((D, D), lambda i: (0, 0))
    b_bs = pl.BlockSpec((1, D), lambda i: (0, 0))
    n1_bs = pl.BlockSpec((N, 1), lambda i: (0, 0))
    out = pl.pallas_call(
        _dense_pool_body,
        grid=(N // _BN,),
        in_specs=[row_bs, agg_bs, w_bs, b_bs, w_bs, b_bs, b_bs, b_bs,
                  n1_bs, w_bs, b_bs, w_bs, b_bs, w_bs, b_bs, w_bs, b_bs],
        out_specs=pl.BlockSpec((G, D), lambda i: (0, 0)),
        out_shape=jax.ShapeDtypeStruct((G, D), jnp.float32),
        scratch_shapes=[pltpu.VMEM((N, D), jnp.float32),
                        pltpu.VMEM((N, 1), jnp.float32)],
    )(hb, ag, wa, ba[None, :], wb, bb[None, :], g[None, :], be[None, :],
      batch[:, None].astype(jnp.int32), Wg1, bg1[None, :], wg2p, bg2p,
      Wc1, bc1[None, :], wc2p, bc2p)
    return out[:, :2]


def kernel(x, edge_index, batch, W0a, b0a, W0b, b0b, ln0_g, ln0_b, W1a, b1a, W1b, b1b, ln1_g, ln1_b, Wg1, bg1, Wg2, bg2, Wc1, bc1, Wc2, bc2):
    eip = jnp.stack([edge_index[0].reshape(NW, NCHUNK, CH),
                     edge_index[1].reshape(NW, NCHUNK, CH)], axis=2)
    agg = _segsum_sc(x, eip)
    h = _dense_layer(x, agg, W0a, b0a, W0b, b0b, ln0_g, ln0_b)
    agg = _segsum_sc(h, eip)
    return _dense_pool(h, agg, W1a, b1a, W1b, b1b, ln1_g, ln1_b, batch,
                       Wg1, bg1, Wg2, bg2, Wc1, bc1, Wc2, bc2)


# speedup vs baseline: 12.7961x; 1.0614x over previous
import functools

import jax
import jax.numpy as jnp
from jax import lax
from jax.experimental import pallas as pl
from jax.experimental.pallas import tpu as pltpu
from jax.experimental.pallas import tpu_sc as plsc

N = 10000
E = 320000
D = 128
G = 16

NC = 2
NS = 16
NW = NC * NS
EPW = E // NW
CH = 80
NCHUNK = EPW // CH
NPAD = 10240
RPT = NPAD // NS
ZR = 16
NB = 3
NI = 6

_sc_mesh = plsc.VectorSubcoreMesh(core_axis_name="c", subcore_axis_name="s")


@functools.partial(
    pl.kernel,
    out_type=jax.ShapeDtypeStruct((NC, NPAD, D), jnp.float32),
    mesh=_sc_mesh,
    scratch_types=[
        [pltpu.VMEM((CH, D), jnp.float32)] * NB,
        [pltpu.VMEM((2, 1, CH), jnp.int32)] * NI,
        pltpu.VMEM((ZR, D), jnp.float32),
        pltpu.VMEM_SHARED((NPAD, D), jnp.float32),
        pltpu.SemaphoreType.DMA((NB,)),
        pltpu.SemaphoreType.DMA((NB,)),
        pltpu.SemaphoreType.DMA((NI,)),
        pltpu.SemaphoreType.DMA,
    ],
)
def _segsum_sc(x_hbm, ei_hbm, out_hbm, bufs, islots, zbuf, acc_sh,
               gsem, ssem, isem, zsem):
    c = lax.axis_index("c")
    s = lax.axis_index("s")
    wid = c * NS + s

    zeros16 = jnp.zeros((16,), jnp.float32)
    for i in range(ZR):
        for j in range(D // 16):
            zbuf[i, pl.ds(j * 16, 16)] = zeros16

    def zero_issue(i, carry):
        pltpu.async_copy(zbuf, acc_sh.at[pl.ds(s * RPT + i * ZR, ZR)], zsem)
        return carry

    lax.fori_loop(0, RPT // ZR, zero_issue, 0)

    def i_issue(k, b):
        pltpu.async_copy(ei_hbm.at[0, wid, k], islots[b].at[0], isem.at[b])
        pltpu.async_copy(ei_hbm.at[1, wid, k], islots[b].at[1], isem.at[b])

    def i_wait(b):
        pltpu.make_async_copy(ei_hbm.at[0, 0, 0], islots[b].at[0],
                              isem.at[b]).wait()
        pltpu.make_async_copy(ei_hbm.at[0, 0, 0], islots[b].at[1],
                              isem.at[b]).wait()

    def g_issue(q, b):
        pltpu.async_copy(x_hbm.at[islots[q].at[0, 0]], bufs[b], gsem.at[b])

    def g_wait(b):
        pltpu.make_async_copy(x_hbm.at[pl.ds(0, CH)], bufs[b],
                              gsem.at[b]).wait()

    def s_issue(b, q):
        pltpu.async_copy(bufs[b], acc_sh.at[islots[q].at[1, 0]], ssem.at[b],
                         add=True)

    def s_wait(b):
        pltpu.make_async_copy(bufs[b], acc_sh.at[pl.ds(0, CH)],
                              ssem.at[b]).wait()

    i_issue(0, 0)
    i_issue(1, 1)
    i_issue(2, 2)

    def zero_drain(i, carry):
        pltpu.make_async_copy(zbuf, acc_sh.at[pl.ds(0, ZR)], zsem).wait()
        return carry

    i_wait(0)
    g_issue(0, 0)
    lax.fori_loop(0, RPT // ZR, zero_drain, 0)
    plsc.subcore_barrier()

    def main_body(i, carry):
        for q in range(NI):
            j = i * NI + q
            p = q % NB
            pn = (p + 1) % NB
            qn = (q + 1) % NI

            i_issue(j + 3, (q + 3) % NI)
            i_wait(qn)
            if q < 2:
                @pl.when(i > 0)
                def _():
                    s_wait(pn)
            else:
                s_wait(pn)
            g_issue(qn, pn)
            g_wait(p)
            s_issue(p, q)
        return carry

    lax.fori_loop(0, (NCHUNK - 5) // NI, main_body, 0)

    def tail_body(i, carry):
        base = i * NI
        i_issue(base + 3, 3)
        i_wait(1)
        s_wait(1)
        g_issue(1, 1)
        g_wait(0)
        s_issue(0, 0)
        i_issue(base + 4, 4)
        i_wait(2)
        s_wait(2)
        g_issue(2, 2)
        g_wait(1)
        s_issue(1, 1)
        i_wait(3)
        s_wait(0)
        g_issue(3, 0)
        g_wait(2)
        s_issue(2, 2)
        i_wait(4)
        s_wait(1)
        g_issue(4, 1)
        g_wait(0)
        s_issue(0, 3)
        s_wait(2)
        g_wait(1)
        s_issue(1, 4)
        return carry

    lax.fori_loop(NCHUNK // NI, NCHUNK // NI + 1, tail_body, 0)
    s_wait(0)
    s_wait(1)

    plsc.subcore_barrier()

    pltpu.sync_copy(acc_sh.at[pl.ds(s * RPT, RPT)],
                    out_hbm.at[c, pl.ds(s * RPT, RPT)])


def _dense_body(hb, ag, wa, ba, wb, bb, g, be, o):
    h = hb[...] + ag[0] + ag[1]
    h = jnp.maximum(jnp.dot(h, wa[...], preferred_element_type=jnp.float32)
                    + ba[...], 0.0)
    h = jnp.dot(h, wb[...], preferred_element_type=jnp.float32) + bb[...]
    mu = jnp.mean(h, axis=-1, keepdims=True)
    var = jnp.mean((h - mu) * (h - mu), axis=-1, keepdims=True)
    h = (h - mu) * lax.rsqrt(var + 1e-5) * g[...] + be[...]
    o[...] = jnp.maximum(h, 0.0)


_BN = 1000


def _dense_layer(hb, ag, wa, ba, wb, bb, g, be):
    row_bs = pl.BlockSpec((_BN, D), lambda i: (i, 0))
    agg_bs = pl.BlockSpec((NC, _BN, D), lambda i: (0, i, 0))
    w_bs = pl.BlockSpec((D, D), lambda i: (0, 0))
    b_bs = pl.BlockSpec((1, D), lambda i: (0, 0))
    return pl.pallas_call(
        _dense_body,
        grid=(N // _BN,),
        in_specs=[row_bs, agg_bs, w_bs, b_bs, w_bs, b_bs, b_bs, b_bs],
        out_specs=row_bs,
        out_shape=jax.ShapeDtypeStruct((N, D), jnp.float32),
    )(hb, ag, wa, ba[None, :], wb, bb[None, :], g[None, :], be[None, :])


def _dense_pool_body(hb, ag, wa, ba, wb, bb, g, be, b_ref, wg1, bg1, wg2,
                     bg2, wc1, bc1, wc2, bc2, o_ref, h_s, gate_s):
    i = pl.program_id(0)
    h = hb[...] + ag[0] + ag[1]
    h = jnp.maximum(jnp.dot(h, wa[...], preferred_element_type=jnp.float32)
                    + ba[...], 0.0)
    h = jnp.dot(h, wb[...], preferred_element_type=jnp.float32) + bb[...]
    mu = jnp.mean(h, axis=-1, keepdims=True)
    var = jnp.mean((h - mu) * (h - mu), axis=-1, keepdims=True)
    h = (h - mu) * lax.rsqrt(var + 1e-5) * g[...] + be[...]
    h = jnp.maximum(h, 0.0)
    h_s[pl.ds(i * _BN, _BN), :] = h
    g1 = jnp.maximum(jnp.dot(h, wg1[...], preferred_element_type=jnp.float32)
                     + bg1[...], 0.0)
    gate_s[pl.ds(i * _BN, _BN), :] = (
        jnp.dot(g1, wg2[...], preferred_element_type=jnp.float32) + bg2[...])

    @pl.when(i == N // _BN - 1)
    def _():
        gate = gate_s[...]
        onehot = b_ref[...] == lax.broadcasted_iota(jnp.int32, (N, G), 1)
        ohf = onehot.astype(jnp.float32)
        gmax = jnp.max(jnp.where(onehot, gate, -3e38), axis=0,
                       keepdims=True)
        gmax_pn = lax.dot_general(ohf, gmax, (((1,), (1,)), ((), ())),
                                  preferred_element_type=jnp.float32)
        e = jnp.exp(gate - gmax_pn)
        den = lax.dot_general(e, ohf, (((0,), (0,)), ((), ())),
                              preferred_element_type=jnp.float32)
        den_pn = lax.dot_general(ohf, den, (((1,), (1,)), ((), ())),
                                 preferred_element_type=jnp.float32)
        w_oh = ohf * (e / den_pn)
        hf = h_s[...]
        pooled_t = lax.dot_general(hf, w_oh, (((0,), (0,)), ((), ())),
                                   preferred_element_type=jnp.float32)
        c1 = lax.dot_general(pooled_t, wc1[...], (((0,), (0,)), ((), ())),
                             preferred_element_type=jnp.float32)
        c1 = jnp.maximum(c1 + bc1[...], 0.0)
        o_ref[...] = (jnp.dot(c1, wc2[...],
                              preferred_element_type=jnp.float32) + bc2[...])


def _dense_pool(hb, ag, wa, ba, wb, bb, g, be, batch,
                Wg1, bg1, Wg2, bg2, Wc1, bc1, Wc2, bc2):
    row_bs = pl.BlockSpec((_BN, D), lambda i: (i, 0))
    agg_bs = pl.BlockSpec((NC, _BN, D), lambda i: (0, i, 0))
    w_bs = pl.BlockSpec((D, D), lambda i: (0, 0))
    b_bs = pl.BlockSpec((1, D), lambda i: (0, 0))
    n1_bs = pl.BlockSpec((N, 1), lambda i: (0, 0))
    c1_bs = pl.BlockSpec((D, 1), lambda i: (0, 0))
    s1_bs = pl.BlockSpec((1, 1), lambda i: (0, 0))
    c2_bs = pl.BlockSpec((D, 2), lambda i: (0, 0))
    s2_bs = pl.BlockSpec((1, 2), lambda i: (0, 0))
    return pl.pallas_call(
        _dense_pool_body,
        grid=(N // _BN,),
        in_specs=[row_bs, agg_bs, w_bs, b_bs, w_bs, b_bs, b_bs, b_bs,
                  n1_bs, w_bs, b_bs, c1_bs, s1_bs, w_bs, b_bs, c2_bs, s2_bs],
        out_specs=pl.BlockSpec((G, 2), lambda i: (0, 0)),
        out_shape=jax.ShapeDtypeStruct((G, 2), jnp.float32),
        scratch_shapes=[pltpu.VMEM((N, D), jnp.float32),
                        pltpu.VMEM((N, 1), jnp.float32)],
    )(hb, ag, wa, ba[None, :], wb, bb[None, :], g[None, :], be[None, :],
      batch[:, None].astype(jnp.int32), Wg1, bg1[None, :], Wg2, bg2[None, :],
      Wc1, bc1[None, :], Wc2, bc2[None, :])


def kernel(x, edge_index, batch, W0a, b0a, W0b, b0b, ln0_g, ln0_b, W1a, b1a, W1b, b1b, ln1_g, ln1_b, Wg1, bg1, Wg2, bg2, Wc1, bc1, Wc2, bc2):
    eip = edge_index.reshape(2, NW, NCHUNK, 1, CH)
    agg = _segsum_sc(x, eip)
    h = _dense_layer(x, agg, W0a, b0a, W0b, b0b, ln0_g, ln0_b)
    agg = _segsum_sc(h, eip)
    return _dense_pool(h, agg, W1a, b1a, W1b, b1b, ln1_g, ln1_b, batch,
                       Wg1, bg1, Wg2, bg2, Wc1, bc1, Wc2, bc2)


# speedup vs baseline: 13.1334x; 1.0264x over previous
import functools

import jax
import jax.numpy as jnp
from jax import lax
from jax.experimental import pallas as pl
from jax.experimental.pallas import tpu as pltpu
from jax.experimental.pallas import tpu_sc as plsc

N = 10000
E = 320000
D = 128
G = 16

NC = 2
NS = 16
NW = NC * NS
EPW = E // NW
CH = 80
NCHUNK = EPW // CH
NPAD = 10240
RPT = NPAD // NS
ZR = 16
NB = 3
NI = 6

_sc_mesh = plsc.VectorSubcoreMesh(core_axis_name="c", subcore_axis_name="s")


@functools.partial(
    pl.kernel,
    out_type=jax.ShapeDtypeStruct((NC, NPAD, D), jnp.float32),
    mesh=_sc_mesh,
    scratch_types=[
        [pltpu.VMEM((CH, D), jnp.float32)] * NB,
        [pltpu.VMEM((2, 1, CH), jnp.int32)] * NI,
        pltpu.VMEM((ZR, D), jnp.float32),
        pltpu.VMEM_SHARED((NPAD, D), jnp.float32),
        pltpu.SemaphoreType.DMA((NB,)),
        pltpu.SemaphoreType.DMA((NB,)),
        pltpu.SemaphoreType.DMA((NI,)),
        pltpu.SemaphoreType.DMA,
    ],
)
def _segsum_sc(x_hbm, ei_hbm, out_hbm, bufs, islots, zbuf, acc_sh,
               gsem, ssem, isem, zsem):
    c = lax.axis_index("c")
    s = lax.axis_index("s")
    wid = c * NS + s

    zeros16 = jnp.zeros((16,), jnp.float32)
    for i in range(ZR):
        for j in range(D // 16):
            zbuf[i, pl.ds(j * 16, 16)] = zeros16

    def zero_issue(i, carry):
        pltpu.async_copy(zbuf, acc_sh.at[pl.ds(s * RPT + i * ZR, ZR)], zsem)
        return carry

    lax.fori_loop(0, RPT // ZR, zero_issue, 0)

    def i_issue(k, b):
        pltpu.async_copy(ei_hbm.at[0, wid, k], islots[b].at[0], isem.at[b])
        pltpu.async_copy(ei_hbm.at[1, wid, k], islots[b].at[1], isem.at[b])

    def i_wait(b):
        pltpu.make_async_copy(ei_hbm.at[0, 0, 0], islots[b].at[0],
                              isem.at[b]).wait()
        pltpu.make_async_copy(ei_hbm.at[0, 0, 0], islots[b].at[1],
                              isem.at[b]).wait()

    def g_issue(q, b):
        pltpu.async_copy(x_hbm.at[islots[q].at[0, 0]], bufs[b], gsem.at[b])

    def g_wait(b):
        pltpu.make_async_copy(x_hbm.at[pl.ds(0, CH)], bufs[b],
                              gsem.at[b]).wait()

    def s_issue(b, q):
        pltpu.async_copy(bufs[b], acc_sh.at[islots[q].at[1, 0]], ssem.at[b],
                         add=True)

    def s_wait(b):
        pltpu.make_async_copy(bufs[b], acc_sh.at[pl.ds(0, CH)],
                              ssem.at[b]).wait()

    i_issue(0, 0)
    i_issue(1, 1)
    i_issue(2, 2)

    def zero_drain(i, carry):
        pltpu.make_async_copy(zbuf, acc_sh.at[pl.ds(0, ZR)], zsem).wait()
        return carry

    i_wait(0)
    g_issue(0, 0)
    lax.fori_loop(0, RPT // ZR, zero_drain, 0)
    plsc.subcore_barrier()

    def main_body(i, carry):
        for q in range(NI):
            j = i * NI + q
            p = q % NB
            pn = (p + 1) % NB
            qn = (q + 1) % NI

            i_issue(j + 3, (q + 3) % NI)
            i_wait(qn)
            if q < 2:
                @pl.when(i > 0)
                def _():
                    s_wait(pn)
            else:
                s_wait(pn)
            g_issue(qn, pn)
            g_wait(p)
            s_issue(p, q)
        return carry

    lax.fori_loop(0, (NCHUNK - 5) // NI, main_body, 0)

    def tail_body(i, carry):
        base = i * NI
        i_issue(base + 3, 3)
        i_wait(1)
        s_wait(1)
        g_issue(1, 1)
        g_wait(0)
        s_issue(0, 0)
        i_issue(base + 4, 4)
        i_wait(2)
        s_wait(2)
        g_issue(2, 2)
        g_wait(1)
        s_issue(1, 1)
        i_wait(3)
        s_wait(0)
        g_issue(3, 0)
        g_wait(2)
        s_issue(2, 2)
        i_wait(4)
        s_wait(1)
        g_issue(4, 1)
        g_wait(0)
        s_issue(0, 3)
        s_wait(2)
        g_wait(1)
        s_issue(1, 4)
        return carry

    lax.fori_loop(NCHUNK // NI, NCHUNK // NI + 1, tail_body, 0)
    s_wait(0)
    s_wait(1)

    plsc.subcore_barrier()

    pltpu.sync_copy(acc_sh.at[pl.ds(s * RPT, RPT)],
                    out_hbm.at[c, pl.ds(s * RPT, RPT)])


def _dense_body(hb, ag, wa, ba, wb, bb, g, be, o):
    h = hb[...] + ag[0] + ag[1]
    h = jnp.maximum(jnp.dot(h, wa[...], preferred_element_type=jnp.float32)
                    + ba[...], 0.0)
    h = jnp.dot(h, wb[...], preferred_element_type=jnp.float32) + bb[...]
    mu = jnp.mean(h, axis=-1, keepdims=True)
    var = jnp.mean((h - mu) * (h - mu), axis=-1, keepdims=True)
    h = (h - mu) * lax.rsqrt(var + 1e-5) * g[...] + be[...]
    o[...] = jnp.maximum(h, 0.0)


_BN = 2000


def _dense_layer(hb, ag, wa, ba, wb, bb, g, be):
    row_bs = pl.BlockSpec((_BN, D), lambda i: (i, 0))
    agg_bs = pl.BlockSpec((NC, _BN, D), lambda i: (0, i, 0))
    w_bs = pl.BlockSpec((D, D), lambda i: (0, 0))
    b_bs = pl.BlockSpec((1, D), lambda i: (0, 0))
    return pl.pallas_call(
        _dense_body,
        grid=(N // _BN,),
        in_specs=[row_bs, agg_bs, w_bs, b_bs, w_bs, b_bs, b_bs, b_bs],
        out_specs=row_bs,
        out_shape=jax.ShapeDtypeStruct((N, D), jnp.float32),
    )(hb, ag, wa, ba[None, :], wb, bb[None, :], g[None, :], be[None, :])


def _dense_pool_body(hb, ag, wa, ba, wb, bb, g, be, b_ref, wg1, bg1, wg2,
                     bg2, wc1, bc1, wc2, bc2, o_ref, h_s, gate_s):
    i = pl.program_id(0)
    h = hb[...] + ag[0] + ag[1]
    h = jnp.maximum(jnp.dot(h, wa[...], preferred_element_type=jnp.float32)
                    + ba[...], 0.0)
    h = jnp.dot(h, wb[...], preferred_element_type=jnp.float32) + bb[...]
    mu = jnp.mean(h, axis=-1, keepdims=True)
    var = jnp.mean((h - mu) * (h - mu), axis=-1, keepdims=True)
    h = (h - mu) * lax.rsqrt(var + 1e-5) * g[...] + be[...]
    h = jnp.maximum(h, 0.0)
    h_s[pl.ds(i * _BN, _BN), :] = h
    g1 = jnp.maximum(jnp.dot(h, wg1[...], preferred_element_type=jnp.float32)
                     + bg1[...], 0.0)
    gate_s[pl.ds(i * _BN, _BN), :] = (
        jnp.dot(g1, wg2[...], preferred_element_type=jnp.float32) + bg2[...])

    @pl.when(i == N // _BN - 1)
    def _():
        gate = gate_s[...]
        onehot = b_ref[...] == lax.broadcasted_iota(jnp.int32, (N, G), 1)
        ohf = onehot.astype(jnp.float32)
        gmax = jnp.max(jnp.where(onehot, gate, -3e38), axis=0,
                       keepdims=True)
        gmax_pn = lax.dot_general(ohf, gmax, (((1,), (1,)), ((), ())),
                                  preferred_element_type=jnp.float32)
        e = jnp.exp(gate - gmax_pn)
        den = lax.dot_general(e, ohf, (((0,), (0,)), ((), ())),
                              preferred_element_type=jnp.float32)
        den_pn = lax.dot_general(ohf, den, (((1,), (1,)), ((), ())),
                                 preferred_element_type=jnp.float32)
        w_oh = ohf * (e / den_pn)
        hf = h_s[...]
        pooled_t = lax.dot_general(hf, w_oh, (((0,), (0,)), ((), ())),
                                   preferred_element_type=jnp.float32)
        c1 = lax.dot_general(pooled_t, wc1[...], (((0,), (0,)), ((), ())),
                             preferred_element_type=jnp.float32)
        c1 = jnp.maximum(c1 + bc1[...], 0.0)
        o_ref[...] = (jnp.dot(c1, wc2[...],
                              preferred_element_type=jnp.float32) + bc2[...])


def _dense_pool(hb, ag, wa, ba, wb, bb, g, be, batch,
                Wg1, bg1, Wg2, bg2, Wc1, bc1, Wc2, bc2):
    row_bs = pl.BlockSpec((_BN, D), lambda i: (i, 0))
    agg_bs = pl.BlockSpec((NC, _BN, D), lambda i: (0, i, 0))
    w_bs = pl.BlockSpec((D, D), lambda i: (0, 0))
    b_bs = pl.BlockSpec((1, D), lambda i: (0, 0))
    n1_bs = pl.BlockSpec((N, 1), lambda i: (0, 0))
    c1_bs = pl.BlockSpec((D, 1), lambda i: (0, 0))
    s1_bs = pl.BlockSpec((1, 1), lambda i: (0, 0))
    c2_bs = pl.BlockSpec((D, 2), lambda i: (0, 0))
    s2_bs = pl.BlockSpec((1, 2), lambda i: (0, 0))
    return pl.pallas_call(
        _dense_pool_body,
        grid=(N // _BN,),
        in_specs=[row_bs, agg_bs, w_bs, b_bs, w_bs, b_bs, b_bs, b_bs,
                  n1_bs, w_bs, b_bs, c1_bs, s1_bs, w_bs, b_bs, c2_bs, s2_bs],
        out_specs=pl.BlockSpec((G, 2), lambda i: (0, 0)),
        out_shape=jax.ShapeDtypeStruct((G, 2), jnp.float32),
        scratch_shapes=[pltpu.VMEM((N, D), jnp.float32),
                        pltpu.VMEM((N, 1), jnp.float32)],
    )(hb, ag, wa, ba[None, :], wb, bb[None, :], g[None, :], be[None, :],
      batch[:, None].astype(jnp.int32), Wg1, bg1[None, :], Wg2, bg2[None, :],
      Wc1, bc1[None, :], Wc2, bc2[None, :])


def kernel(x, edge_index, batch, W0a, b0a, W0b, b0b, ln0_g, ln0_b, W1a, b1a, W1b, b1b, ln1_g, ln1_b, Wg1, bg1, Wg2, bg2, Wc1, bc1, Wc2, bc2):
    eip = edge_index.reshape(2, NW, NCHUNK, 1, CH)
    agg = _segsum_sc(x, eip)
    h = _dense_layer(x, agg, W0a, b0a, W0b, b0b, ln0_g, ln0_b)
    agg = _segsum_sc(h, eip)
    return _dense_pool(h, agg, W1a, b1a, W1b, b1b, ln1_g, ln1_b, batch,
                       Wg1, bg1, Wg2, bg2, Wc1, bc1, Wc2, bc2)
